# baseline (speedup 1.0000x reference)
import functools

import jax
import jax.numpy as jnp
from jax import lax
from jax.experimental import pallas as pl
from jax.experimental.pallas import tpu as pltpu
from jax.experimental.pallas import tpu_sc as plsc

N = 10000
E = 320000
DS = 128
DV = 32

NC = 2
NS = 16
CH = 80


def _dotT(x, w):
    return lax.dot_general(x, w, (((1,), (1,)), ((), ())),
                           preferred_element_type=jnp.float32)


def _dot(x, w):
    return lax.dot_general(x, w, (((1,), (0,)), ((), ())),
                           preferred_element_type=jnp.float32)


def _proj_body(s_ref, wa_ref, wb_ref, sa_ref, sb_ref):
    x = s_ref[...]
    sa_ref[...] = _dotT(x, wa_ref[...])
    sb_ref[...] = _dotT(x, wb_ref[...])


def _node_proj(s, W1a, W1b):
    B = 1000
    return pl.pallas_call(
        _proj_body,
        grid=(N // B,),
        in_specs=[pl.BlockSpec((B, DS), lambda i: (i, 0)),
                  pl.BlockSpec((DS, DS), lambda i: (0, 0)),
                  pl.BlockSpec((DS, DS), lambda i: (0, 0))],
        out_specs=[pl.BlockSpec((B, DS), lambda i: (i, 0)),
                   pl.BlockSpec((B, DS), lambda i: (i, 0))],
        out_shape=[jax.ShapeDtypeStruct((N, DS), jnp.float32),
                   jax.ShapeDtypeStruct((N, DS), jnp.float32)],
    )(s, W1a, W1b)


def _gather(sa, sb, row, col):
    mesh = plsc.VectorSubcoreMesh(core_axis_name="c", subcore_axis_name="s")
    per_w = E // (NC * NS)
    n_ch = per_w // CH

    @functools.partial(
        pl.kernel, mesh=mesh,
        out_type=jax.ShapeDtypeStruct((E, DS), jnp.float32),
        scratch_types=[
            pltpu.VMEM((per_w,), jnp.int32),
            pltpu.VMEM((per_w,), jnp.int32),
            pltpu.VMEM((CH, DS), jnp.float32),
            pltpu.VMEM((CH, DS), jnp.float32),
            pltpu.SemaphoreType.DMA,
            pltpu.SemaphoreType.DMA,
        ],
    )
    def k(sa_hbm, sb_hbm, row_hbm, col_hbm, g_hbm,
          idxr, idxc, bufa, bufb, sem_a, sem_b):
        cid = lax.axis_index("c")
        sid = lax.axis_index("s")
        wid = sid * NC + cid
        base = wid * per_w
        pltpu.sync_copy(row_hbm.at[pl.ds(base, per_w)], idxr)
        pltpu.sync_copy(col_hbm.at[pl.ds(base, per_w)], idxc)

        def body(i, carry):
            eb = i * CH
            cpa = pltpu.async_copy(sa_hbm.at[idxr.at[pl.ds(eb, CH)]], bufa, sem_a)
            cpb = pltpu.async_copy(sb_hbm.at[idxc.at[pl.ds(eb, CH)]], bufb, sem_b)
            cpa.wait()
            cpb.wait()

            def add_row(r, c2):
                for c8 in range(DS // 16):
                    sl = pl.ds(c8 * 16, 16)
                    bufa[r, sl] = bufa[r, sl] + bufb[r, sl]
                return c2
            lax.fori_loop(0, CH, add_row, 0, unroll=2)
            pltpu.sync_copy(bufa, g_hbm.at[pl.ds(base + eb, CH)])
            return carry
        lax.fori_loop(0, n_ch, body, 0)

    return k(sa, sb, row, col)


def _edge_body(g_ref, ea_ref, un_ref, w1c_ref, b1_ref, w2_ref, b2_ref,
               wvb_ref, bvb_ref, p_ref, m_ref, w_ref):
    h = g_ref[...] + _dotT(ea_ref[...], w1c_ref[...]) + b1_ref[...]
    h = h * jax.nn.sigmoid(h)
    m = _dotT(h, w2_ref[...]) + b2_ref[...]
    m_ref[...] = m
    sm = m * jax.nn.sigmoid(m)
    w = _dot(sm, wvb_ref[...]) + bvb_ref[...]
    u = _dot(un_ref[...], p_ref[...])
    w_ref[...] = w * u


def _edge_mlp(g, ea, unit, W1c, b1, W2, b2, Wv_big, bv_big, P):
    B = 512
    return pl.pallas_call(
        _edge_body,
        grid=(E // B,),
        in_specs=[pl.BlockSpec((B, DS), lambda i: (i, 0)),
                  pl.BlockSpec((B, DS), lambda i: (i, 0)),
                  pl.BlockSpec((B, 3), lambda i: (i, 0)),
                  pl.BlockSpec((DS, DS), lambda i: (0, 0)),
                  pl.BlockSpec((1, DS), lambda i: (0, 0)),
                  pl.BlockSpec((DS, DS), lambda i: (0, 0)),
                  pl.BlockSpec((1, DS), lambda i: (0, 0)),
                  pl.BlockSpec((DS, DS), lambda i: (0, 0)),
                  pl.BlockSpec((1, DS), lambda i: (0, 0)),
                  pl.BlockSpec((3, DS), lambda i: (0, 0))],
        out_specs=[pl.BlockSpec((B, DS), lambda i: (i, 0)),
                   pl.BlockSpec((B, DS), lambda i: (i, 0))],
        out_shape=[jax.ShapeDtypeStruct((E, DS), jnp.float32),
                   jax.ShapeDtypeStruct((E, DS), jnp.float32)],
    )(g, ea, unit, W1c, b1, W2, b2, Wv_big, bv_big, P)


NP = 10240


def _scatter(m, w, row, v2p):
    mesh = plsc.VectorSubcoreMesh(core_axis_name="c", subcore_axis_name="s")
    per_t = E // NS
    n_ch = per_t // CH
    nb = NP // NS
    ZR = 64

    @functools.partial(
        pl.kernel, mesh=mesh,
        out_type=[jax.ShapeDtypeStruct((NP, DS), jnp.float32),
                  jax.ShapeDtypeStruct((NP, 96), jnp.float32)],
        scratch_types=[
            pltpu.VMEM_SHARED((NP, DS), jnp.float32),
            pltpu.VMEM((CH,), jnp.int32),
            pltpu.VMEM((CH, DS), jnp.float32),
            pltpu.VMEM((ZR, DS), jnp.float32),
            pltpu.VMEM((ZR, 96), jnp.float32),
            pltpu.VMEM((ZR, 96), jnp.float32),
        ],
    )
    def k(m_hbm, w_hbm, row_hbm, v_hbm, sout_hbm, vnew_hbm,
          acc, idxb, datab, stage, vstage, vnstage):
        cid = lax.axis_index("c")
        sid = lax.axis_index("s")
        zero16 = jnp.zeros((16,), jnp.float32)

        def zrow(r, c):
            for c8 in range(DS // 16):
                stage[r, pl.ds(c8 * 16, 16)] = zero16
            return c
        lax.fori_loop(0, ZR, zrow, 0)
        nbase = sid * nb
        for j in range(nb // ZR):
            pltpu.sync_copy(stage, acc.at[pl.ds(nbase + j * ZR, ZR)])
        plsc.subcore_barrier()

        ebase = sid * per_t

        def sc_body(i, c):
            eb = ebase + i * CH
            pltpu.sync_copy(row_hbm.at[pl.ds(eb, CH)], idxb)

            @pl.when(cid == 0)
            def _():
                pltpu.sync_copy(m_hbm.at[pl.ds(eb, CH)], datab)

            @pl.when(cid == 1)
            def _():
                pltpu.sync_copy(w_hbm.at[pl.ds(eb, CH)], datab)

            pltpu.sync_copy(datab, acc.at[idxb], add=True)
            return c
        lax.fori_loop(0, n_ch, sc_body, 0)
        plsc.subcore_barrier()

        @pl.when(cid == 0)
        def _():
            for j in range(nb // ZR):
                b0 = nbase + j * ZR
                pltpu.sync_copy(acc.at[pl.ds(b0, ZR)], stage)
                pltpu.sync_copy(stage, sout_hbm.at[pl.ds(b0, ZR)])

        @pl.when(cid == 1)
        def _():
            for j in range(nb // ZR):
                b0 = nbase + j * ZR
                pltpu.sync_copy(acc.at[pl.ds(b0, ZR)], stage)
                pltpu.sync_copy(v_hbm.at[pl.ds(b0, ZR)], vstage)

                def prow(r, c):
                    for vv in range(6):
                        sl = pl.ds(vv * 16, 16)
                        vnstage[r, sl] = vstage[r, sl] + stage[r, sl]
                    return c
                lax.fori_loop(0, ZR, prow, 0)
                pltpu.sync_copy(vnstage, vnew_hbm.at[pl.ds(b0, ZR)])

    return k(m, w, row, v2p)


def _node_body(s_ref, so_ref, ws_ref, bs_ref, g_ref, b_ref, out_ref):
    so = so_ref[...]
    sm = so * jax.nn.sigmoid(so)
    u = s_ref[...] + _dotT(sm, ws_ref[...]) + bs_ref[...]
    mu = jnp.mean(u, axis=1, keepdims=True)
    d = u - mu
    var = jnp.mean(d * d, axis=1, keepdims=True)
    out_ref[...] = d * lax.rsqrt(var + 1e-5) * g_ref[...] + b_ref[...]


def _node_out(s, s_out, W_s, bs, g, b):
    B = 1000
    return pl.pallas_call(
        _node_body,
        grid=(N // B,),
        in_specs=[pl.BlockSpec((B, DS), lambda i: (i, 0)),
                  pl.BlockSpec((B, DS), lambda i: (i, 0)),
                  pl.BlockSpec((DS, DS), lambda i: (0, 0)),
                  pl.BlockSpec((1, DS), lambda i: (0, 0)),
                  pl.BlockSpec((1, DS), lambda i: (0, 0)),
                  pl.BlockSpec((1, DS), lambda i: (0, 0))],
        out_specs=pl.BlockSpec((B, DS), lambda i: (i, 0)),
        out_shape=jax.ShapeDtypeStruct((N, DS), jnp.float32),
    )(s, s_out, W_s, bs, g, b)


def kernel(s, v, edge_index, edge_attr, edge_vec_unit,
           W_e1, b_e1, W_e2, b_e2, W_s, b_s, W_v, b_v, ln_g, ln_b):
    row = edge_index[0]
    col = edge_index[1]
    W1a = W_e1[:, :DS]
    W1b = W_e1[:, DS:2 * DS]
    W1c = W_e1[:, 2 * DS:]
    WvT = W_v.T
    jidx = jnp.arange(96) // 3
    Wv_int = jnp.concatenate(
        [WvT[:, jidx], jnp.zeros((DS, DV), jnp.float32)], axis=1)
    bv_int = jnp.pad(b_v[jidx], (0, DV)).reshape(1, DS)
    kcol = jnp.arange(96) % 3
    P = jnp.pad((kcol[None, :] == jnp.arange(3)[:, None]).astype(jnp.float32),
                ((0, 0), (0, DV)))

    sa, sb = _node_proj(s, W1a, W1b)
    g = _gather(sa, sb, row, col)
    m, w = _edge_mlp(g, edge_attr, edge_vec_unit, W1c,
                     b_e1.reshape(1, DS), W_e2, b_e2.reshape(1, DS),
                     Wv_int, bv_int, P)
    v2p = jnp.pad(v.reshape(N, 3 * DV), ((0, NP - N), (0, 0)))
    s_out_p, v_new96p = _scatter(m, w, row, v2p)
    s_new = _node_out(s, s_out_p[:N], W_s, b_s.reshape(1, DS),
                      ln_g.reshape(1, DS), ln_b.reshape(1, DS))
    return (s_new, v_new96p[:N].reshape(N, DV, 3))


# speedup vs baseline: 1.2944x; 1.2944x over previous
import functools

import jax
import jax.numpy as jnp
from jax import lax
from jax.experimental import pallas as pl
from jax.experimental.pallas import tpu as pltpu
from jax.experimental.pallas import tpu_sc as plsc

N = 10000
E = 320000
DS = 128
DV = 32

NC = 2
NS = 16
CH = 80

NP = 10240
EP = 4096 * CH
CB = CH * DS * 4


def _dotT(x, w):
    return lax.dot_general(x, w, (((1,), (1,)), ((), ())),
                           preferred_element_type=jnp.float32)


def _dot(x, w):
    return lax.dot_general(x, w, (((1,), (0,)), ((), ())),
                           preferred_element_type=jnp.float32)


def _proj_body(s_ref, wa_ref, wb_ref, sa_ref, sb_ref):
    x = s_ref[...]
    sa_ref[...] = _dotT(x, wa_ref[...])
    sb_ref[...] = _dotT(x, wb_ref[...])


def _node_proj(s, W1a, W1b):
    B = 1000
    return pl.pallas_call(
        _proj_body,
        grid=(N // B,),
        in_specs=[pl.BlockSpec((B, DS), lambda i: (i, 0)),
                  pl.BlockSpec((DS, DS), lambda i: (0, 0)),
                  pl.BlockSpec((DS, DS), lambda i: (0, 0))],
        out_specs=[pl.BlockSpec((B, DS), lambda i: (i, 0)),
                   pl.BlockSpec((B, DS), lambda i: (i, 0))],
        out_shape=[jax.ShapeDtypeStruct((N, DS), jnp.float32),
                   jax.ShapeDtypeStruct((N, DS), jnp.float32)],
    )(s, W1a, W1b)


def _gather(sa, sb, row, col):
    mesh = plsc.VectorSubcoreMesh(core_axis_name="c", subcore_axis_name="s")
    per_w = E // (NC * NS)
    n_ch = per_w // CH
    n_pair = (n_ch - 1) // 2

    @functools.partial(
        pl.kernel, mesh=mesh,
        out_type=jax.ShapeDtypeStruct((E, DS), jnp.float32),
        scratch_types=[
            pltpu.VMEM((per_w,), jnp.int32),
            pltpu.VMEM((per_w,), jnp.int32),
            pltpu.VMEM((CH, DS), jnp.float32),
            pltpu.VMEM((CH, DS), jnp.float32),
            pltpu.VMEM((CH, DS), jnp.float32),
            pltpu.VMEM((CH, DS), jnp.float32),
            pltpu.SemaphoreType.DMA,
            pltpu.SemaphoreType.DMA,
            pltpu.SemaphoreType.DMA,
            pltpu.SemaphoreType.DMA,
        ],
    )
    def k(sa_hbm, sb_hbm, row_hbm, col_hbm, g_hbm,
          idxr, idxc, a0, b0, a1, b1, in0, in1, wr0, wr1):
        cid = lax.axis_index("c")
        sid = lax.axis_index("s")
        wid = sid * NC + cid
        base = wid * per_w
        pltpu.sync_copy(row_hbm.at[pl.ds(base, per_w)], idxr)
        pltpu.sync_copy(col_hbm.at[pl.ds(base, per_w)], idxc)

        def issue(i, abuf, bbuf, sem):
            pltpu.async_copy(sa_hbm.at[idxr.at[pl.ds(i * CH, CH)]], abuf, sem)
            pltpu.async_copy(sb_hbm.at[idxc.at[pl.ds(i * CH, CH)]], bbuf, sem)

        def wait_in(abuf, bbuf, sem):
            pltpu.make_async_copy(sa_hbm.at[pl.ds(0, CH)], abuf, sem).wait()
            pltpu.make_async_copy(sa_hbm.at[pl.ds(0, CH)], bbuf, sem).wait()

        def wait_wr(abuf, sem):
            pltpu.make_async_copy(abuf, g_hbm.at[pl.ds(0, CH)], sem).wait()

        def add_write(i, abuf, bbuf, sem_w):
            def add_row(r, c2):
                for c8 in range(DS // 16):
                    sl = pl.ds(c8 * 16, 16)
                    abuf[r, sl] = abuf[r, sl] + bbuf[r, sl]
                return c2
            lax.fori_loop(0, CH, add_row, 0, unroll=4)
            pltpu.async_copy(abuf, g_hbm.at[pl.ds(base + i * CH, CH)], sem_w)

        issue(0, a0, b0, in0)

        def body(p, carry):
            c0 = 2 * p

            @pl.when(p > 0)
            def _():
                wait_wr(a1, wr1)
            issue(c0 + 1, a1, b1, in1)
            wait_in(a0, b0, in0)
            add_write(c0, a0, b0, wr0)
            wait_wr(a0, wr0)
            issue(c0 + 2, a0, b0, in0)
            wait_in(a1, b1, in1)
            add_write(c0 + 1, a1, b1, wr1)
            return carry
        lax.fori_loop(0, n_pair, body, 0)

        wait_in(a0, b0, in0)
        add_write(n_ch - 1, a0, b0, wr0)
        wait_wr(a0, wr0)
        wait_wr(a1, wr1)

    return k(sa, sb, row, col)


def _edge_body(g_ref, ea_ref, un_ref, w1c_ref, b1_ref, w2_ref, b2_ref,
               wvb_ref, bvb_ref, p_ref, m_ref, w_ref):
    h = g_ref[...] + _dotT(ea_ref[...], w1c_ref[...]) + b1_ref[...]
    h = h * jax.nn.sigmoid(h)
    m = _dotT(h, w2_ref[...]) + b2_ref[...]
    sm = m * jax.nn.sigmoid(m)
    w = _dot(sm, wvb_ref[...]) + bvb_ref[...]
    u = _dot(un_ref[...], p_ref[...])
    live = pl.program_id(0) < (E // 512)
    m_ref[...] = jnp.where(live, m, 0.0)
    w_ref[...] = jnp.where(live, w * u, 0.0)


def _edge_mlp(g, ea, unit, W1c, b1, W2, b2, Wv_int, bv_int, P):
    B = 512
    nlive = E // B
    clamp = lambda i: (jnp.minimum(i, nlive - 1), 0)
    const = lambda i: (0, 0)
    return pl.pallas_call(
        _edge_body,
        grid=(EP // B,),
        in_specs=[pl.BlockSpec((B, DS), clamp),
                  pl.BlockSpec((B, DS), clamp),
                  pl.BlockSpec((B, 3), clamp),
                  pl.BlockSpec((DS, DS), const),
                  pl.BlockSpec((1, DS), const),
                  pl.BlockSpec((DS, DS), const),
                  pl.BlockSpec((1, DS), const),
                  pl.BlockSpec((DS, DS), const),
                  pl.BlockSpec((1, DS), const),
                  pl.BlockSpec((3, DS), const)],
        out_specs=[pl.BlockSpec((B, DS), lambda i: (i, 0)),
                   pl.BlockSpec((B, DS), lambda i: (i, 0))],
        out_shape=[jax.ShapeDtypeStruct((EP, DS), jnp.float32),
                   jax.ShapeDtypeStruct((EP, DS), jnp.float32)],
    )(g, ea, unit, W1c, b1, W2, b2, Wv_int, bv_int, P)


def _scatter(m, w, row2d, v2p):
    mesh = plsc.VectorSubcoreMesh(core_axis_name="c", subcore_axis_name="s")
    n_rows = EP // CH
    per_t = n_rows // NS
    HC = per_t // 2
    nb = NP // NS
    ZR = 32

    @functools.partial(
        pl.kernel, mesh=mesh,
        out_type=[jax.ShapeDtypeStruct((NP, DS), jnp.float32),
                  jax.ShapeDtypeStruct((NP, 96), jnp.float32)],
        scratch_types=[
            pltpu.VMEM_SHARED((NP, DS), jnp.float32),
            pltpu.VMEM((HC, CH), jnp.int32),
            pltpu.VMEM((CH, DS), jnp.float32),
            pltpu.VMEM((CH, DS), jnp.float32),
            pltpu.VMEM((ZR, DS), jnp.float32),
            pltpu.VMEM((ZR, 96), jnp.float32),
            pltpu.VMEM((ZR, 96), jnp.float32),
            pltpu.SemaphoreType.DMA,
            pltpu.SemaphoreType.DMA,
            pltpu.SemaphoreType.DMA,
            pltpu.SemaphoreType.DMA,
        ],
    )
    def k(m_hbm, w_hbm, row_hbm, v_hbm, sout_hbm, vnew_hbm,
          acc, idxs, dat0, dat1, stage, vstage, vnstage,
          in0, in1, ss0, ss1):
        cid = lax.axis_index("c")
        sid = lax.axis_index("s")
        zero16 = jnp.zeros((16,), jnp.float32)

        def zrow(r, c):
            for c8 in range(DS // 16):
                stage[r, pl.ds(c8 * 16, 16)] = zero16
            return c
        lax.fori_loop(0, ZR, zrow, 0)
        nbase = sid * nb
        for j in range(nb // ZR):
            pltpu.sync_copy(stage, acc.at[pl.ds(nbase + j * ZR, ZR)])
        plsc.subcore_barrier()

        def issue(gr, dat, sem):
            @pl.when(cid == 0)
            def _():
                pltpu.async_copy(m_hbm.at[pl.ds(gr * CH, CH)], dat, sem)

            @pl.when(cid == 1)
            def _():
                pltpu.async_copy(w_hbm.at[pl.ds(gr * CH, CH)], dat, sem)

        def wait_in(dat, sem):
            pltpu.make_async_copy(m_hbm.at[pl.ds(0, CH)], dat, sem).wait()

        def wait_ss(dat, sem):
            pltpu.make_async_copy(dat, acc.at[idxs.at[0]], sem).wait()

        cbase = sid * per_t
        for h in range(2):
            hbase = cbase + h * HC
            pltpu.sync_copy(row_hbm.at[pl.ds(hbase, HC)], idxs)
            issue(hbase, dat0, in0)

            def body(p, carry):
                c0 = 2 * p

                @pl.when(p > 0)
                def _():
                    wait_ss(dat1, ss1)
                issue(hbase + c0 + 1, dat1, in1)
                wait_in(dat0, in0)
                pltpu.async_copy(dat0, acc.at[idxs.at[c0]], ss0, add=True)
                wait_ss(dat0, ss0)

                @pl.when(p < HC // 2 - 1)
                def _():
                    issue(hbase + c0 + 2, dat0, in0)
                wait_in(dat1, in1)
                pltpu.async_copy(dat1, acc.at[idxs.at[c0 + 1]], ss1, add=True)
                return carry
            lax.fori_loop(0, HC // 2, body, 0)
            wait_ss(dat1, ss1)
        plsc.subcore_barrier()

        @pl.when(cid == 0)
        def _():
            for j in range(nb // ZR):
                b0 = nbase + j * ZR
                pltpu.sync_copy(acc.at[pl.ds(b0, ZR)], stage)
                pltpu.sync_copy(stage, sout_hbm.at[pl.ds(b0, ZR)])

        @pl.when(cid == 1)
        def _():
            for j in range(nb // ZR):
                b0 = nbase + j * ZR
                pltpu.sync_copy(acc.at[pl.ds(b0, ZR)], stage)
                pltpu.sync_copy(v_hbm.at[pl.ds(b0, ZR)], vstage)

                def prow(r, c):
                    for vv in range(6):
                        sl = pl.ds(vv * 16, 16)
                        vnstage[r, sl] = vstage[r, sl] + stage[r, sl]
                    return c
                lax.fori_loop(0, ZR, prow, 0, unroll=4)
                pltpu.sync_copy(vnstage, vnew_hbm.at[pl.ds(b0, ZR)])

    return k(m, w, row2d, v2p)


def _node_body(s_ref, so_ref, ws_ref, bs_ref, g_ref, b_ref, out_ref):
    so = so_ref[...]
    sm = so * jax.nn.sigmoid(so)
    u = s_ref[...] + _dotT(sm, ws_ref[...]) + bs_ref[...]
    mu = jnp.mean(u, axis=1, keepdims=True)
    d = u - mu
    var = jnp.mean(d * d, axis=1, keepdims=True)
    out_ref[...] = d * lax.rsqrt(var + 1e-5) * g_ref[...] + b_ref[...]


def _node_out(s, s_out, W_s, bs, g, b):
    B = 1000
    return pl.pallas_call(
        _node_body,
        grid=(N // B,),
        in_specs=[pl.BlockSpec((B, DS), lambda i: (i, 0)),
                  pl.BlockSpec((B, DS), lambda i: (i, 0)),
                  pl.BlockSpec((DS, DS), lambda i: (0, 0)),
                  pl.BlockSpec((1, DS), lambda i: (0, 0)),
                  pl.BlockSpec((1, DS), lambda i: (0, 0)),
                  pl.BlockSpec((1, DS), lambda i: (0, 0))],
        out_specs=pl.BlockSpec((B, DS), lambda i: (i, 0)),
        out_shape=jax.ShapeDtypeStruct((N, DS), jnp.float32),
    )(s, s_out, W_s, bs, g, b)


def kernel(s, v, edge_index, edge_attr, edge_vec_unit,
           W_e1, b_e1, W_e2, b_e2, W_s, b_s, W_v, b_v, ln_g, ln_b):
    row = edge_index[0]
    col = edge_index[1]
    W1a = W_e1[:, :DS]
    W1b = W_e1[:, DS:2 * DS]
    W1c = W_e1[:, 2 * DS:]
    WvT = W_v.T
    jidx = jnp.arange(96) // 3
    Wv_int = jnp.concatenate(
        [WvT[:, jidx], jnp.zeros((DS, DV), jnp.float32)], axis=1)
    bv_int = jnp.pad(b_v[jidx], (0, DV)).reshape(1, DS)
    kcol = jnp.arange(96) % 3
    P = jnp.pad((kcol[None, :] == jnp.arange(3)[:, None]).astype(jnp.float32),
                ((0, 0), (0, DV)))

    sa, sb = _node_proj(s, W1a, W1b)
    g = _gather(sa, sb, row, col)
    m, w = _edge_mlp(g, edge_attr, edge_vec_unit, W1c,
                     b_e1.reshape(1, DS), W_e2, b_e2.reshape(1, DS),
                     Wv_int, bv_int, P)
    row2d = jnp.pad(row, (0, EP - E), mode="wrap").reshape(EP // CH, CH)
    v2p = jnp.pad(v.reshape(N, 3 * DV), ((0, NP - N), (0, 0)))
    s_out_p, v_new96p = _scatter(m, w, row2d, v2p)
    s_new = _node_out(s, s_out_p[:N], W_s, b_s.reshape(1, DS),
                      ln_g.reshape(1, DS), ln_b.reshape(1, DS))
    return (s_new, v_new96p[:N].reshape(N, DV, 3))


# speedup vs baseline: 1.6073x; 1.2417x over previous
import functools

import jax
import jax.numpy as jnp
from jax import lax
from jax.experimental import pallas as pl
from jax.experimental.pallas import tpu as pltpu
from jax.experimental.pallas import tpu_sc as plsc

N = 10000
E = 320000
DS = 128
DV = 32

NC = 2
NS = 16
CH = 80

NP = 10240
EP = 4096 * CH
CB = CH * DS * 4


def _dotT(x, w):
    return lax.dot_general(x, w, (((1,), (1,)), ((), ())),
                           preferred_element_type=jnp.float32)


def _dot(x, w):
    return lax.dot_general(x, w, (((1,), (0,)), ((), ())),
                           preferred_element_type=jnp.float32)


def _proj_body(s_ref, wa_ref, wb_ref, sa_ref, sb_ref):
    x = s_ref[...]
    sa_ref[...] = _dotT(x, wa_ref[...])
    sb_ref[...] = _dotT(x, wb_ref[...])


def _node_proj(s, W1a, W1b):
    B = 1000
    return pl.pallas_call(
        _proj_body,
        grid=(N // B,),
        in_specs=[pl.BlockSpec((B, DS), lambda i: (i, 0)),
                  pl.BlockSpec((DS, DS), lambda i: (0, 0)),
                  pl.BlockSpec((DS, DS), lambda i: (0, 0))],
        out_specs=[pl.BlockSpec((B, DS), lambda i: (i, 0)),
                   pl.BlockSpec((B, DS), lambda i: (i, 0))],
        out_shape=[jax.ShapeDtypeStruct((N, DS), jnp.float32),
                   jax.ShapeDtypeStruct((N, DS), jnp.float32)],
    )(s, W1a, W1b)


def _gather(sa, sb, row, col):
    mesh = plsc.VectorSubcoreMesh(core_axis_name="c", subcore_axis_name="s")
    per_w = E // (NC * NS)
    n_ch = per_w // CH
    n_tri = (n_ch - 2) // 3

    @functools.partial(
        pl.kernel, mesh=mesh,
        out_type=jax.ShapeDtypeStruct((E, DS), jnp.float32),
        scratch_types=[
            pltpu.VMEM((per_w,), jnp.int32),
            pltpu.VMEM((per_w,), jnp.int32),
            [pltpu.VMEM((CH, DS), jnp.float32) for _ in range(3)],
            [pltpu.VMEM((CH, DS), jnp.float32) for _ in range(3)],
            [pltpu.SemaphoreType.DMA for _ in range(3)],
            [pltpu.SemaphoreType.DMA for _ in range(3)],
        ],
    )
    def k(sa_hbm, sb_hbm, row_hbm, col_hbm, g_hbm,
          idxr, idxc, abufs, bbufs, insems, wrsems):
        cid = lax.axis_index("c")
        sid = lax.axis_index("s")
        wid = sid * NC + cid
        base = wid * per_w
        pltpu.sync_copy(row_hbm.at[pl.ds(base, per_w)], idxr)
        pltpu.sync_copy(col_hbm.at[pl.ds(base, per_w)], idxc)

        def issue(i, r):
            pltpu.async_copy(sa_hbm.at[idxr.at[pl.ds(i * CH, CH)]],
                             abufs[r], insems[r])
            pltpu.async_copy(sb_hbm.at[idxc.at[pl.ds(i * CH, CH)]],
                             bbufs[r], insems[r])

        def wait_in(r):
            pltpu.make_async_copy(sa_hbm.at[pl.ds(0, CH)], abufs[r],
                                  insems[r]).wait()
            pltpu.make_async_copy(sa_hbm.at[pl.ds(0, CH)], bbufs[r],
                                  insems[r]).wait()

        def wait_wr(r):
            pltpu.make_async_copy(abufs[r], g_hbm.at[pl.ds(0, CH)],
                                  wrsems[r]).wait()

        def add_write(i, r):
            abuf, bbuf = abufs[r], bbufs[r]

            def add_row(rr, c2):
                for c8 in range(DS // 16):
                    sl = pl.ds(c8 * 16, 16)
                    abuf[rr, sl] = abuf[rr, sl] + bbuf[rr, sl]
                return c2
            lax.fori_loop(0, CH, add_row, 0, unroll=4)
            pltpu.async_copy(abuf, g_hbm.at[pl.ds(base + i * CH, CH)],
                             wrsems[r])

        issue(0, 0)
        issue(1, 1)

        def body(t, carry):
            c = 3 * t
            for kk in range(3):
                r = kk
                rn = (kk + 2) % 3
                wait_in(r)
                add_write(c + kk, r)
                if kk == 0:
                    @pl.when(t > 0)
                    def _():
                        wait_wr(rn)
                else:
                    wait_wr(rn)
                issue(c + kk + 2, rn)
            return carry
        lax.fori_loop(0, n_tri, body, 0)

        wait_in(0)
        add_write(n_ch - 2, 0)
        wait_in(1)
        add_write(n_ch - 1, 1)
        for r in range(3):
            wait_wr(r)

    return k(sa, sb, row, col)


def _edge_body(g_ref, ea_ref, un_ref, w1c_ref, b1_ref, w2_ref, b2_ref,
               wvb_ref, bvb_ref, p_ref, m_ref, w_ref):
    h = g_ref[...] + _dotT(ea_ref[...], w1c_ref[...]) + b1_ref[...]
    h = h * jax.nn.sigmoid(h)
    m = _dotT(h, w2_ref[...]) + b2_ref[...]
    sm = m * jax.nn.sigmoid(m)
    w = _dot(sm, wvb_ref[...]) + bvb_ref[...]
    u = _dot(un_ref[...], p_ref[...])
    B = m.shape[0]
    erow = pl.program_id(0) * B + lax.broadcasted_iota(jnp.int32, (B, 1), 0)
    live = erow < E
    m_ref[...] = jnp.where(live, m, 0.0)
    w_ref[...] = jnp.where(live, w * u, 0.0)


def _edge_mlp(g, ea, unit, W1c, b1, W2, b2, Wv_int, bv_int, P):
    B = 1024
    nlive = (E + B - 1) // B
    clamp = lambda i: (jnp.minimum(i, nlive - 1), 0)
    const = lambda i: (0, 0)
    return pl.pallas_call(
        _edge_body,
        grid=(EP // B,),
        in_specs=[pl.BlockSpec((B, DS), clamp),
                  pl.BlockSpec((B, DS), clamp),
                  pl.BlockSpec((B, 3), clamp),
                  pl.BlockSpec((DS, DS), const),
                  pl.BlockSpec((1, DS), const),
                  pl.BlockSpec((DS, DS), const),
                  pl.BlockSpec((1, DS), const),
                  pl.BlockSpec((DS, DS), const),
                  pl.BlockSpec((1, DS), const),
                  pl.BlockSpec((3, DS), const)],
        out_specs=[pl.BlockSpec((B, DS), lambda i: (i, 0)),
                   pl.BlockSpec((B, DS), lambda i: (i, 0))],
        out_shape=[jax.ShapeDtypeStruct((EP, DS), jnp.float32),
                   jax.ShapeDtypeStruct((EP, DS), jnp.float32)],
    )(g, ea, unit, W1c, b1, W2, b2, Wv_int, bv_int, P)


def _scatter(m, w, row2d, v2p):
    mesh = plsc.VectorSubcoreMesh(core_axis_name="c", subcore_axis_name="s")
    n_rows = EP // CH
    per_t = n_rows // NS
    HC = per_t // 2
    nb = NP // NS
    ZR = 32

    @functools.partial(
        pl.kernel, mesh=mesh,
        out_type=[jax.ShapeDtypeStruct((NP, DS), jnp.float32),
                  jax.ShapeDtypeStruct((NP, 96), jnp.float32)],
        scratch_types=[
            pltpu.VMEM_SHARED((NP, DS), jnp.float32),
            pltpu.VMEM((HC, CH), jnp.int32),
            pltpu.VMEM((CH, DS), jnp.float32),
            pltpu.VMEM((CH, DS), jnp.float32),
            pltpu.VMEM((ZR, DS), jnp.float32),
            pltpu.VMEM((ZR, 96), jnp.float32),
            pltpu.VMEM((ZR, 96), jnp.float32),
            pltpu.SemaphoreType.DMA,
            pltpu.SemaphoreType.DMA,
            pltpu.SemaphoreType.DMA,
            pltpu.SemaphoreType.DMA,
        ],
    )
    def k(m_hbm, w_hbm, row_hbm, v_hbm, sout_hbm, vnew_hbm,
          acc, idxs, dat0, dat1, stage, vstage, vnstage,
          in0, in1, ss0, ss1):
        cid = lax.axis_index("c")
        sid = lax.axis_index("s")
        zero16 = jnp.zeros((16,), jnp.float32)

        def zrow(r, c):
            for c8 in range(DS // 16):
                stage[r, pl.ds(c8 * 16, 16)] = zero16
            return c
        lax.fori_loop(0, ZR, zrow, 0)
        nbase = sid * nb
        for j in range(nb // ZR):
            pltpu.sync_copy(stage, acc.at[pl.ds(nbase + j * ZR, ZR)])
        plsc.subcore_barrier()

        def issue(gr, dat, sem):
            @pl.when(cid == 0)
            def _():
                pltpu.async_copy(m_hbm.at[pl.ds(gr * CH, CH)], dat, sem)

            @pl.when(cid == 1)
            def _():
                pltpu.async_copy(w_hbm.at[pl.ds(gr * CH, CH)], dat, sem)

        def wait_in(dat, sem):
            pltpu.make_async_copy(m_hbm.at[pl.ds(0, CH)], dat, sem).wait()

        def wait_ss(dat, sem):
            pltpu.make_async_copy(dat, acc.at[idxs.at[0]], sem).wait()

        cbase = sid * per_t
        for h in range(2):
            hbase = cbase + h * HC
            pltpu.sync_copy(row_hbm.at[pl.ds(hbase, HC)], idxs)
            issue(hbase, dat0, in0)

            def body(p, carry):
                c0 = 2 * p

                @pl.when(p > 0)
                def _():
                    wait_ss(dat1, ss1)
                issue(hbase + c0 + 1, dat1, in1)
                wait_in(dat0, in0)
                pltpu.async_copy(dat0, acc.at[idxs.at[c0]], ss0, add=True)
                wait_ss(dat0, ss0)

                @pl.when(p < HC // 2 - 1)
                def _():
                    issue(hbase + c0 + 2, dat0, in0)
                wait_in(dat1, in1)
                pltpu.async_copy(dat1, acc.at[idxs.at[c0 + 1]], ss1, add=True)
                return carry
            lax.fori_loop(0, HC // 2, body, 0)
            wait_ss(dat1, ss1)
        plsc.subcore_barrier()

        @pl.when(cid == 0)
        def _():
            for j in range(nb // ZR):
                b0 = nbase + j * ZR
                pltpu.sync_copy(acc.at[pl.ds(b0, ZR)], stage)
                pltpu.sync_copy(stage, sout_hbm.at[pl.ds(b0, ZR)])

        @pl.when(cid == 1)
        def _():
            for j in range(nb // ZR):
                b0 = nbase + j * ZR
                pltpu.sync_copy(acc.at[pl.ds(b0, ZR)], stage)
                pltpu.sync_copy(v_hbm.at[pl.ds(b0, ZR)], vstage)

                def prow(r, c):
                    for vv in range(6):
                        sl = pl.ds(vv * 16, 16)
                        vnstage[r, sl] = vstage[r, sl] + stage[r, sl]
                    return c
                lax.fori_loop(0, ZR, prow, 0, unroll=4)
                pltpu.sync_copy(vnstage, vnew_hbm.at[pl.ds(b0, ZR)])

    return k(m, w, row2d, v2p)


def _node_body(s_ref, so_ref, ws_ref, bs_ref, g_ref, b_ref, out_ref):
    so = so_ref[...]
    sm = so * jax.nn.sigmoid(so)
    u = s_ref[...] + _dotT(sm, ws_ref[...]) + bs_ref[...]
    mu = jnp.mean(u, axis=1, keepdims=True)
    d = u - mu
    var = jnp.mean(d * d, axis=1, keepdims=True)
    out_ref[...] = d * lax.rsqrt(var + 1e-5) * g_ref[...] + b_ref[...]


def _node_out(s, s_out, W_s, bs, g, b):
    B = 1000
    return pl.pallas_call(
        _node_body,
        grid=(N // B,),
        in_specs=[pl.BlockSpec((B, DS), lambda i: (i, 0)),
                  pl.BlockSpec((B, DS), lambda i: (i, 0)),
                  pl.BlockSpec((DS, DS), lambda i: (0, 0)),
                  pl.BlockSpec((1, DS), lambda i: (0, 0)),
                  pl.BlockSpec((1, DS), lambda i: (0, 0)),
                  pl.BlockSpec((1, DS), lambda i: (0, 0))],
        out_specs=pl.BlockSpec((B, DS), lambda i: (i, 0)),
        out_shape=jax.ShapeDtypeStruct((N, DS), jnp.float32),
    )(s, s_out, W_s, bs, g, b)


def kernel(s, v, edge_index, edge_attr, edge_vec_unit,
           W_e1, b_e1, W_e2, b_e2, W_s, b_s, W_v, b_v, ln_g, ln_b):
    row = edge_index[0]
    col = edge_index[1]
    W1a = W_e1[:, :DS]
    W1b = W_e1[:, DS:2 * DS]
    W1c = W_e1[:, 2 * DS:]
    WvT = W_v.T
    jidx = jnp.arange(96) // 3
    Wv_int = jnp.concatenate(
        [WvT[:, jidx], jnp.zeros((DS, DV), jnp.float32)], axis=1)
    bv_int = jnp.pad(b_v[jidx], (0, DV)).reshape(1, DS)
    kcol = jnp.arange(96) % 3
    P = jnp.pad((kcol[None, :] == jnp.arange(3)[:, None]).astype(jnp.float32),
                ((0, 0), (0, DV)))

    sa, sb = _node_proj(s, W1a, W1b)
    g = _gather(sa, sb, row, col)
    m, w = _edge_mlp(g, edge_attr, edge_vec_unit, W1c,
                     b_e1.reshape(1, DS), W_e2, b_e2.reshape(1, DS),
                     Wv_int, bv_int, P)
    row2d = jnp.pad(row, (0, EP - E), mode="wrap").reshape(EP // CH, CH)
    v2p = jnp.pad(v.reshape(N, 3 * DV), ((0, NP - N), (0, 0)))
    s_out_p, v_new96p = _scatter(m, w, row2d, v2p)
    s_new = _node_out(s, s_out_p[:N], W_s, b_s.reshape(1, DS),
                      ln_g.reshape(1, DS), ln_b.reshape(1, DS))
    return (s_new, v_new96p[:N].reshape(N, DV, 3))


# speedup vs baseline: 1.6535x; 1.0287x over previous
import functools

import jax
import jax.numpy as jnp
from jax import lax
from jax.experimental import pallas as pl
from jax.experimental.pallas import tpu as pltpu
from jax.experimental.pallas import tpu_sc as plsc

N = 10000
E = 320000
DS = 128
DV = 32

NC = 2
NS = 16
CH = 80

NP = 10240
EP = 4096 * CH
CB = CH * DS * 4


def _dotT(x, w):
    return lax.dot_general(x, w, (((1,), (1,)), ((), ())),
                           preferred_element_type=jnp.float32)


def _dot(x, w):
    return lax.dot_general(x, w, (((1,), (0,)), ((), ())),
                           preferred_element_type=jnp.float32)


def _proj_body(s_ref, wa_ref, wb_ref, sa_ref, sb_ref):
    x = s_ref[...]
    sa_ref[...] = _dotT(x, wa_ref[...])
    sb_ref[...] = _dotT(x, wb_ref[...])


def _node_proj(s, W1a, W1b):
    B = 1000
    return pl.pallas_call(
        _proj_body,
        grid=(N // B,),
        in_specs=[pl.BlockSpec((B, DS), lambda i: (i, 0)),
                  pl.BlockSpec((DS, DS), lambda i: (0, 0)),
                  pl.BlockSpec((DS, DS), lambda i: (0, 0))],
        out_specs=[pl.BlockSpec((B, DS), lambda i: (i, 0)),
                   pl.BlockSpec((B, DS), lambda i: (i, 0))],
        out_shape=[jax.ShapeDtypeStruct((N, DS), jnp.float32),
                   jax.ShapeDtypeStruct((N, DS), jnp.float32)],
    )(s, W1a, W1b)


def _gather(sa, sb, row, col):
    mesh = plsc.VectorSubcoreMesh(core_axis_name="c", subcore_axis_name="s")
    per_w = E // (NC * NS)
    n_ch = per_w // CH
    n_quad = (n_ch - 1) // 4
    NR = 4

    @functools.partial(
        pl.kernel, mesh=mesh,
        out_type=jax.ShapeDtypeStruct((E, DS), jnp.float32),
        scratch_types=[
            pltpu.VMEM((per_w,), jnp.int32),
            pltpu.VMEM((per_w,), jnp.int32),
            [pltpu.VMEM((CH, DS), jnp.float32) for _ in range(NR)],
            [pltpu.VMEM((CH, DS), jnp.float32) for _ in range(NR)],
            [pltpu.SemaphoreType.DMA for _ in range(NR)],
            [pltpu.SemaphoreType.DMA for _ in range(NR)],
        ],
    )
    def k(sa_hbm, sb_hbm, row_hbm, col_hbm, g_hbm,
          idxr, idxc, abufs, bbufs, insems, wrsems):
        cid = lax.axis_index("c")
        sid = lax.axis_index("s")
        wid = sid * NC + cid
        base = wid * per_w
        pltpu.sync_copy(row_hbm.at[pl.ds(base, per_w)], idxr)
        pltpu.sync_copy(col_hbm.at[pl.ds(base, per_w)], idxc)

        def issue(i, r):
            pltpu.async_copy(sa_hbm.at[idxr.at[pl.ds(i * CH, CH)]],
                             abufs[r], insems[r])
            pltpu.async_copy(sb_hbm.at[idxc.at[pl.ds(i * CH, CH)]],
                             bbufs[r], insems[r])

        def wait_in(r):
            pltpu.make_async_copy(sa_hbm.at[pl.ds(0, CH)], abufs[r],
                                  insems[r]).wait()
            pltpu.make_async_copy(sa_hbm.at[pl.ds(0, CH)], bbufs[r],
                                  insems[r]).wait()

        def wait_wr(r):
            pltpu.make_async_copy(abufs[r], g_hbm.at[pl.ds(0, CH)],
                                  wrsems[r]).wait()

        def add_write(i, r):
            abuf, bbuf = abufs[r], bbufs[r]

            def add_row(rr, c2):
                for c8 in range(DS // 16):
                    sl = pl.ds(c8 * 16, 16)
                    abuf[rr, sl] = abuf[rr, sl] + bbuf[rr, sl]
                return c2
            lax.fori_loop(0, CH, add_row, 0, unroll=4)
            pltpu.async_copy(abuf, g_hbm.at[pl.ds(base + i * CH, CH)],
                             wrsems[r])

        for r in range(NR - 1):
            issue(r, r)

        def body(t, carry):
            c = NR * t
            for kk in range(NR):
                r = kk
                rn = (kk + NR - 1) % NR
                wait_in(r)
                add_write(c + kk, r)
                ck = c + kk

                @pl.when((ck > 0) & (ck + NR - 1 < n_ch))
                def _():
                    wait_wr(rn)

                @pl.when(ck + NR - 1 < n_ch)
                def _():
                    issue(ck + NR - 1, rn)
            return carry
        lax.fori_loop(0, n_quad, body, 0)

        wait_in(0)
        add_write(n_ch - 1, 0)
        for r in range(NR):
            wait_wr(r)

    return k(sa, sb, row, col)


def _edge_body(g_ref, ea_ref, un_ref, w1c_ref, b1_ref, w2_ref, b2_ref,
               wvb_ref, bvb_ref, p_ref, m_ref, w_ref):
    h = g_ref[...] + _dotT(ea_ref[...], w1c_ref[...]) + b1_ref[...]
    h = h * jax.nn.sigmoid(h)
    m = _dotT(h, w2_ref[...]) + b2_ref[...]
    sm = m * jax.nn.sigmoid(m)
    w = _dot(sm, wvb_ref[...]) + bvb_ref[...]
    u = _dot(un_ref[...], p_ref[...])
    B = m.shape[0]
    erow = pl.program_id(0) * B + lax.broadcasted_iota(jnp.int32, (B, 1), 0)
    live = erow < E
    m_ref[...] = jnp.where(live, m, 0.0)
    w_ref[...] = jnp.where(live, w * u, 0.0)


def _edge_mlp(g, ea, unit, W1c, b1, W2, b2, Wv_int, bv_int, P):
    B = 1024
    nlive = (E + B - 1) // B
    clamp = lambda i: (jnp.minimum(i, nlive - 1), 0)
    const = lambda i: (0, 0)
    return pl.pallas_call(
        _edge_body,
        grid=(EP // B,),
        in_specs=[pl.BlockSpec((B, DS), clamp),
                  pl.BlockSpec((B, DS), clamp),
                  pl.BlockSpec((B, 3), clamp),
                  pl.BlockSpec((DS, DS), const),
                  pl.BlockSpec((1, DS), const),
                  pl.BlockSpec((DS, DS), const),
                  pl.BlockSpec((1, DS), const),
                  pl.BlockSpec((DS, DS), const),
                  pl.BlockSpec((1, DS), const),
                  pl.BlockSpec((3, DS), const)],
        out_specs=[pl.BlockSpec((B, DS), lambda i: (i, 0)),
                   pl.BlockSpec((B, DS), lambda i: (i, 0))],
        out_shape=[jax.ShapeDtypeStruct((EP, DS), jnp.float32),
                   jax.ShapeDtypeStruct((EP, DS), jnp.float32)],
    )(g, ea, unit, W1c, b1, W2, b2, Wv_int, bv_int, P)


def _scatter(m, w, rowp, v2p):
    mesh = plsc.VectorSubcoreMesh(core_axis_name="c", subcore_axis_name="s")
    n_rows = EP // CH
    per_t = n_rows // NS
    n_tri = (per_t - 1) // 3
    nb = NP // NS
    ZR = 16
    NR = 3

    @functools.partial(
        pl.kernel, mesh=mesh,
        out_type=[jax.ShapeDtypeStruct((NP, DS), jnp.float32),
                  jax.ShapeDtypeStruct((NP, 96), jnp.float32)],
        scratch_types=[
            pltpu.VMEM_SHARED((NP, DS), jnp.float32),
            pltpu.VMEM((NR, CH), jnp.int32),
            [pltpu.VMEM((CH, DS), jnp.float32) for _ in range(NR)],
            pltpu.VMEM((ZR, DS), jnp.float32),
            pltpu.VMEM((ZR, 96), jnp.float32),
            pltpu.VMEM((ZR, 96), jnp.float32),
            [pltpu.SemaphoreType.DMA for _ in range(NR)],
            [pltpu.SemaphoreType.DMA for _ in range(NR)],
        ],
    )
    def k(m_hbm, w_hbm, row_hbm, v_hbm, sout_hbm, vnew_hbm,
          acc, idxs, dats, stage, vstage, vnstage, insems, sssems):
        cid = lax.axis_index("c")
        sid = lax.axis_index("s")
        zero16 = jnp.zeros((16,), jnp.float32)

        def zrow(r, c):
            for c8 in range(DS // 16):
                stage[r, pl.ds(c8 * 16, 16)] = zero16
            return c
        lax.fori_loop(0, ZR, zrow, 0)
        nbase = sid * nb

        def zblk(j, c):
            b0 = pl.multiple_of(nbase + j * ZR, 8)
            pltpu.sync_copy(stage, acc.at[pl.ds(b0, ZR)])
            return c
        lax.fori_loop(0, nb // ZR, zblk, 0)
        plsc.subcore_barrier()

        cbase = sid * per_t

        def issue(lc, r):
            gr = cbase + lc
            pltpu.async_copy(row_hbm.at[pl.ds(gr * CH, CH)], idxs.at[r],
                             insems[r])

            @pl.when(cid == 0)
            def _():
                pltpu.async_copy(m_hbm.at[pl.ds(gr * CH, CH)], dats[r],
                                 insems[r])

            @pl.when(cid == 1)
            def _():
                pltpu.async_copy(w_hbm.at[pl.ds(gr * CH, CH)], dats[r],
                                 insems[r])

        def wait_in(r):
            pltpu.make_async_copy(row_hbm.at[pl.ds(0, CH)], idxs.at[r],
                                  insems[r]).wait()
            pltpu.make_async_copy(m_hbm.at[pl.ds(0, CH)], dats[r],
                                  insems[r]).wait()

        def scat(r):
            pltpu.async_copy(dats[r], acc.at[idxs.at[r]], sssems[r], add=True)

        def wait_ss(r):
            pltpu.make_async_copy(dats[r], acc.at[idxs.at[r]],
                                  sssems[r]).wait()

        for r in range(NR - 1):
            issue(r, r)

        def body(t, carry):
            c = NR * t
            for kk in range(NR):
                r = kk
                rn = (kk + NR - 1) % NR
                wait_in(r)
                scat(r)
                ck = c + kk

                @pl.when((ck > 0) & (ck + NR - 1 < per_t))
                def _():
                    wait_ss(rn)

                @pl.when(ck + NR - 1 < per_t)
                def _():
                    issue(ck + NR - 1, rn)
            return carry
        lax.fori_loop(0, n_tri, body, 0)

        wait_in(0)
        scat(0)
        for r in range(NR):
            wait_ss(r)
        plsc.subcore_barrier()

        @pl.when(cid == 0)
        def _():
            def sblk(j, c):
                b0 = pl.multiple_of(nbase + j * ZR, 8)
                pltpu.sync_copy(acc.at[pl.ds(b0, ZR)], stage)
                pltpu.sync_copy(stage, sout_hbm.at[pl.ds(b0, ZR)])
                return c
            lax.fori_loop(0, nb // ZR, sblk, 0)

        @pl.when(cid == 1)
        def _():
            def vblk(j, c):
                b0 = pl.multiple_of(nbase + j * ZR, 8)
                pltpu.sync_copy(acc.at[pl.ds(b0, ZR)], stage)
                pltpu.sync_copy(v_hbm.at[pl.ds(b0, ZR)], vstage)

                def prow(r, c2):
                    for vv in range(6):
                        sl = pl.ds(vv * 16, 16)
                        vnstage[r, sl] = vstage[r, sl] + stage[r, sl]
                    return c2
                lax.fori_loop(0, ZR, prow, 0, unroll=4)
                pltpu.sync_copy(vnstage, vnew_hbm.at[pl.ds(b0, ZR)])
                return c
            lax.fori_loop(0, nb // ZR, vblk, 0)

    return k(m, w, rowp, v2p)


def _node_body(s_ref, so_ref, ws_ref, bs_ref, g_ref, b_ref, out_ref):
    so = so_ref[...]
    sm = so * jax.nn.sigmoid(so)
    u = s_ref[...] + _dotT(sm, ws_ref[...]) + bs_ref[...]
    mu = jnp.mean(u, axis=1, keepdims=True)
    d = u - mu
    var = jnp.mean(d * d, axis=1, keepdims=True)
    out_ref[...] = d * lax.rsqrt(var + 1e-5) * g_ref[...] + b_ref[...]


def _node_out(s, s_out, W_s, bs, g, b):
    B = 1000
    return pl.pallas_call(
        _node_body,
        grid=(N // B,),
        in_specs=[pl.BlockSpec((B, DS), lambda i: (i, 0)),
                  pl.BlockSpec((B, DS), lambda i: (i, 0)),
                  pl.BlockSpec((DS, DS), lambda i: (0, 0)),
                  pl.BlockSpec((1, DS), lambda i: (0, 0)),
                  pl.BlockSpec((1, DS), lambda i: (0, 0)),
                  pl.BlockSpec((1, DS), lambda i: (0, 0))],
        out_specs=pl.BlockSpec((B, DS), lambda i: (i, 0)),
        out_shape=jax.ShapeDtypeStruct((N, DS), jnp.float32),
    )(s, s_out, W_s, bs, g, b)


def kernel(s, v, edge_index, edge_attr, edge_vec_unit,
           W_e1, b_e1, W_e2, b_e2, W_s, b_s, W_v, b_v, ln_g, ln_b):
    row = edge_index[0]
    col = edge_index[1]
    W1a = W_e1[:, :DS]
    W1b = W_e1[:, DS:2 * DS]
    W1c = W_e1[:, 2 * DS:]
    WvT = W_v.T
    jidx = jnp.arange(96) // 3
    Wv_int = jnp.concatenate(
        [WvT[:, jidx], jnp.zeros((DS, DV), jnp.float32)], axis=1)
    bv_int = jnp.pad(b_v[jidx], (0, DV)).reshape(1, DS)
    kcol = jnp.arange(96) % 3
    P = jnp.pad((kcol[None, :] == jnp.arange(3)[:, None]).astype(jnp.float32),
                ((0, 0), (0, DV)))

    sa, sb = _node_proj(s, W1a, W1b)
    g = _gather(sa, sb, row, col)
    m, w = _edge_mlp(g, edge_attr, edge_vec_unit, W1c,
                     b_e1.reshape(1, DS), W_e2, b_e2.reshape(1, DS),
                     Wv_int, bv_int, P)
    rowp = jnp.pad(row, (0, EP - E), mode="wrap")
    v2p = jnp.pad(v.reshape(N, 3 * DV), ((0, NP - N), (0, 0)))
    s_out_p, v_new96p = _scatter(m, w, rowp, v2p)
    s_new = _node_out(s, s_out_p[:N], W_s, b_s.reshape(1, DS),
                      ln_g.reshape(1, DS), ln_b.reshape(1, DS))
    return (s_new, v_new96p[:N].reshape(N, DV, 3))


# speedup vs baseline: 1.8245x; 1.1034x over previous
import functools

import jax
import jax.numpy as jnp
from jax import lax
from jax.experimental import pallas as pl
from jax.experimental.pallas import tpu as pltpu
from jax.experimental.pallas import tpu_sc as plsc

N = 10000
E = 320000
DS = 128
DV = 32

NC = 2
NS = 16
CH = 80

NP = 10240
EP = 4096 * CH
E0 = 204800
E1 = E - E0
EPB = EP - E0


def _dotT(x, w):
    return lax.dot_general(x, w, (((1,), (1,)), ((), ())),
                           preferred_element_type=jnp.float32)


def _dot(x, w):
    return lax.dot_general(x, w, (((1,), (0,)), ((), ())),
                           preferred_element_type=jnp.float32)


def _proj_body(s_ref, wa_ref, wb_ref, sa_ref, sb_ref):
    x = s_ref[...]
    sa_ref[...] = _dotT(x, wa_ref[...])
    sb_ref[...] = _dotT(x, wb_ref[...])


def _node_proj(s, W1a, W1b):
    B = 1000
    return pl.pallas_call(
        _proj_body,
        grid=(N // B,),
        in_specs=[pl.BlockSpec((B, DS), lambda i: (i, 0)),
                  pl.BlockSpec((DS, DS), lambda i: (0, 0)),
                  pl.BlockSpec((DS, DS), lambda i: (0, 0))],
        out_specs=[pl.BlockSpec((B, DS), lambda i: (i, 0)),
                   pl.BlockSpec((B, DS), lambda i: (i, 0))],
        out_shape=[jax.ShapeDtypeStruct((N, DS), jnp.float32),
                   jax.ShapeDtypeStruct((N, DS), jnp.float32)],
    )(s, W1a, W1b)


def _gather(sa, sb, row, col, e0, esz):
    mesh = plsc.VectorSubcoreMesh(core_axis_name="c", subcore_axis_name="s")
    per_w = esz // (NC * NS)
    n_ch = per_w // CH
    n_quad = n_ch // 4
    n_tail = n_ch % 4
    NR = 4

    @functools.partial(
        pl.kernel, mesh=mesh,
        out_type=jax.ShapeDtypeStruct((esz, DS), jnp.float32),
        scratch_types=[
            pltpu.VMEM((per_w,), jnp.int32),
            pltpu.VMEM((per_w,), jnp.int32),
            [pltpu.VMEM((CH, DS), jnp.float32) for _ in range(NR)],
            [pltpu.VMEM((CH, DS), jnp.float32) for _ in range(NR)],
            [pltpu.SemaphoreType.DMA for _ in range(NR)],
            [pltpu.SemaphoreType.DMA for _ in range(NR)],
        ],
    )
    def k(sa_hbm, sb_hbm, row_hbm, col_hbm, g_hbm,
          idxr, idxc, abufs, bbufs, insems, wrsems):
        cid = lax.axis_index("c")
        sid = lax.axis_index("s")
        wid = sid * NC + cid
        base = wid * per_w
        pltpu.sync_copy(row_hbm.at[pl.ds(e0 + base, per_w)], idxr)
        pltpu.sync_copy(col_hbm.at[pl.ds(e0 + base, per_w)], idxc)

        def issue(i, r):
            pltpu.async_copy(sa_hbm.at[idxr.at[pl.ds(i * CH, CH)]],
                             abufs[r], insems[r])
            pltpu.async_copy(sb_hbm.at[idxc.at[pl.ds(i * CH, CH)]],
                             bbufs[r], insems[r])

        def wait_in(r):
            pltpu.make_async_copy(sa_hbm.at[pl.ds(0, CH)], abufs[r],
                                  insems[r]).wait()
            pltpu.make_async_copy(sa_hbm.at[pl.ds(0, CH)], bbufs[r],
                                  insems[r]).wait()

        def wait_wr(r):
            pltpu.make_async_copy(abufs[r], g_hbm.at[pl.ds(0, CH)],
                                  wrsems[r]).wait()

        def add_write(i, r):
            abuf, bbuf = abufs[r], bbufs[r]

            def add_row(rr, c2):
                for c8 in range(DS // 16):
                    sl = pl.ds(c8 * 16, 16)
                    abuf[rr, sl] = abuf[rr, sl] + bbuf[rr, sl]
                return c2
            lax.fori_loop(0, CH, add_row, 0, unroll=4)
            pltpu.async_copy(abuf, g_hbm.at[pl.ds(base + i * CH, CH)],
                             wrsems[r])

        for r in range(NR - 1):
            issue(r, r)

        def body(t, carry):
            c = NR * t
            for kk in range(NR):
                r = kk
                rn = (kk + NR - 1) % NR
                wait_in(r)
                add_write(c + kk, r)
                ck = c + kk

                @pl.when((ck > 0) & (ck + NR - 1 < n_ch))
                def _():
                    wait_wr(rn)

                @pl.when(ck + NR - 1 < n_ch)
                def _():
                    issue(ck + NR - 1, rn)
            return carry
        lax.fori_loop(0, n_quad, body, 0)

        for ck in range(n_quad * NR, n_ch):
            wait_in(ck % NR)
            add_write(ck, ck % NR)
        for r in range(NR):
            wait_wr(r)

    return k(sa, sb, row, col)


def _edge_body_sz(nlive, g_ref, ea_ref, un_ref, w1c_ref, b1_ref, w2_ref,
                  b2_ref, wvb_ref, bvb_ref, p_ref, m_ref, w_ref):
    h = g_ref[...] + _dotT(ea_ref[...], w1c_ref[...]) + b1_ref[...]
    h = h * jax.nn.sigmoid(h)
    m = _dotT(h, w2_ref[...]) + b2_ref[...]
    sm = m * jax.nn.sigmoid(m)
    w = _dot(sm, wvb_ref[...]) + bvb_ref[...]
    u = _dot(un_ref[...], p_ref[...])
    B = m.shape[0]
    erow = pl.program_id(0) * B + lax.broadcasted_iota(jnp.int32, (B, 1), 0)
    live = erow < nlive
    m_ref[...] = jnp.where(live, m, 0.0)
    w_ref[...] = jnp.where(live, w * u, 0.0)


def _edge_mlp(g, ea, unit, W1c, b1, W2, b2, Wv_int, bv_int, P, blk0, esz, epad):
    B = 1024
    nlive_blk = (esz - 1) // B
    clampg = lambda i: (jnp.minimum(i, nlive_blk), 0)
    clampo = lambda i: (jnp.minimum(i, nlive_blk) + blk0, 0)
    const = lambda i: (0, 0)
    body = functools.partial(_edge_body_sz, esz)
    return pl.pallas_call(
        body,
        grid=(epad // B,),
        in_specs=[pl.BlockSpec((B, DS), clampg),
                  pl.BlockSpec((B, DS), clampo),
                  pl.BlockSpec((B, 3), clampo),
                  pl.BlockSpec((DS, DS), const),
                  pl.BlockSpec((1, DS), const),
                  pl.BlockSpec((DS, DS), const),
                  pl.BlockSpec((1, DS), const),
                  pl.BlockSpec((DS, DS), const),
                  pl.BlockSpec((1, DS), const),
                  pl.BlockSpec((3, DS), const)],
        out_specs=[pl.BlockSpec((B, DS), lambda i: (i, 0)),
                   pl.BlockSpec((B, DS), lambda i: (i, 0))],
        out_shape=[jax.ShapeDtypeStruct((epad, DS), jnp.float32),
                   jax.ShapeDtypeStruct((epad, DS), jnp.float32)],
    )(g, ea, unit, W1c, b1, W2, b2, Wv_int, bv_int, P)


def _scatter(m0, w0, m1, w1, rowp, v2p):
    mesh = plsc.VectorSubcoreMesh(core_axis_name="c", subcore_axis_name="s")
    n_rows = EP // CH
    per_t = n_rows // NS
    n_tri = (per_t - 1) // 3
    nb = NP // NS
    ZR = 16
    NR = 3

    @functools.partial(
        pl.kernel, mesh=mesh,
        out_type=[jax.ShapeDtypeStruct((NP, DS), jnp.float32),
                  jax.ShapeDtypeStruct((NP, 96), jnp.float32)],
        scratch_types=[
            pltpu.VMEM_SHARED((NP, DS), jnp.float32),
            pltpu.VMEM((NR, CH), jnp.int32),
            [pltpu.VMEM((CH, DS), jnp.float32) for _ in range(NR)],
            pltpu.VMEM((ZR, DS), jnp.float32),
            pltpu.VMEM((ZR, 96), jnp.float32),
            pltpu.VMEM((ZR, 96), jnp.float32),
            [pltpu.SemaphoreType.DMA for _ in range(NR)],
            [pltpu.SemaphoreType.DMA for _ in range(NR)],
        ],
    )
    def k(m0_hbm, w0_hbm, m1_hbm, w1_hbm, row_hbm, v_hbm, sout_hbm, vnew_hbm,
          acc, idxs, dats, stage, vstage, vnstage, insems, sssems):
        cid = lax.axis_index("c")
        sid = lax.axis_index("s")
        zero16 = jnp.zeros((16,), jnp.float32)

        def zrow(r, c):
            for c8 in range(DS // 16):
                stage[r, pl.ds(c8 * 16, 16)] = zero16
            return c
        lax.fori_loop(0, ZR, zrow, 0)
        nbase = sid * nb

        def zblk(j, c):
            b0 = pl.multiple_of(nbase + j * ZR, 8)
            pltpu.sync_copy(stage, acc.at[pl.ds(b0, ZR)])
            return c
        lax.fori_loop(0, nb // ZR, zblk, 0)
        plsc.subcore_barrier()

        cbase = sid * per_t

        half1 = sid >= (E0 // (CH * 256))

        def issue(lc, r):
            gr = cbase + lc
            pltpu.async_copy(row_hbm.at[pl.ds(gr * CH, CH)], idxs.at[r],
                             insems[r])
            g1 = (gr - E0 // CH) * CH

            @pl.when((cid == 0) & jnp.logical_not(half1))
            def _():
                pltpu.async_copy(m0_hbm.at[pl.ds(gr * CH, CH)], dats[r],
                                 insems[r])

            @pl.when((cid == 1) & jnp.logical_not(half1))
            def _():
                pltpu.async_copy(w0_hbm.at[pl.ds(gr * CH, CH)], dats[r],
                                 insems[r])

            @pl.when((cid == 0) & half1)
            def _():
                pltpu.async_copy(m1_hbm.at[pl.ds(g1, CH)], dats[r],
                                 insems[r])

            @pl.when((cid == 1) & half1)
            def _():
                pltpu.async_copy(w1_hbm.at[pl.ds(g1, CH)], dats[r],
                                 insems[r])

        def wait_in(r):
            pltpu.make_async_copy(row_hbm.at[pl.ds(0, CH)], idxs.at[r],
                                  insems[r]).wait()
            pltpu.make_async_copy(m0_hbm.at[pl.ds(0, CH)], dats[r],
                                  insems[r]).wait()

        def scat(r):
            pltpu.async_copy(dats[r], acc.at[idxs.at[r]], sssems[r], add=True)

        def wait_ss(r):
            pltpu.make_async_copy(dats[r], acc.at[idxs.at[r]],
                                  sssems[r]).wait()

        for r in range(NR - 1):
            issue(r, r)

        def body(t, carry):
            c = NR * t
            for kk in range(NR):
                r = kk
                rn = (kk + NR - 1) % NR
                wait_in(r)
                scat(r)
                ck = c + kk

                @pl.when((ck > 0) & (ck + NR - 1 < per_t))
                def _():
                    wait_ss(rn)

                @pl.when(ck + NR - 1 < per_t)
                def _():
                    issue(ck + NR - 1, rn)
            return carry
        lax.fori_loop(0, n_tri, body, 0)

        wait_in(0)
        scat(0)
        for r in range(NR):
            wait_ss(r)
        plsc.subcore_barrier()

        @pl.when(cid == 0)
        def _():
            def sblk(j, c):
                b0 = pl.multiple_of(nbase + j * ZR, 8)
                pltpu.sync_copy(acc.at[pl.ds(b0, ZR)], stage)
                pltpu.sync_copy(stage, sout_hbm.at[pl.ds(b0, ZR)])
                return c
            lax.fori_loop(0, nb // ZR, sblk, 0)

        @pl.when(cid == 1)
        def _():
            def vblk(j, c):
                b0 = pl.multiple_of(nbase + j * ZR, 8)
                pltpu.sync_copy(acc.at[pl.ds(b0, ZR)], stage)
                pltpu.sync_copy(v_hbm.at[pl.ds(b0, ZR)], vstage)

                def prow(r, c2):
                    for vv in range(6):
                        sl = pl.ds(vv * 16, 16)
                        vnstage[r, sl] = vstage[r, sl] + stage[r, sl]
                    return c2
                lax.fori_loop(0, ZR, prow, 0, unroll=4)
                pltpu.sync_copy(vnstage, vnew_hbm.at[pl.ds(b0, ZR)])
                return c
            lax.fori_loop(0, nb // ZR, vblk, 0)

    return k(m0, w0, m1, w1, rowp, v2p)


def _node_body(s_ref, so_ref, ws_ref, bs_ref, g_ref, b_ref, out_ref):
    so = so_ref[...]
    sm = so * jax.nn.sigmoid(so)
    u = s_ref[...] + _dotT(sm, ws_ref[...]) + bs_ref[...]
    mu = jnp.mean(u, axis=1, keepdims=True)
    d = u - mu
    var = jnp.mean(d * d, axis=1, keepdims=True)
    out_ref[...] = d * lax.rsqrt(var + 1e-5) * g_ref[...] + b_ref[...]


def _node_out(s, s_out, W_s, bs, g, b):
    B = 1000
    return pl.pallas_call(
        _node_body,
        grid=(N // B,),
        in_specs=[pl.BlockSpec((B, DS), lambda i: (i, 0)),
                  pl.BlockSpec((B, DS), lambda i: (i, 0)),
                  pl.BlockSpec((DS, DS), lambda i: (0, 0)),
                  pl.BlockSpec((1, DS), lambda i: (0, 0)),
                  pl.BlockSpec((1, DS), lambda i: (0, 0)),
                  pl.BlockSpec((1, DS), lambda i: (0, 0))],
        out_specs=pl.BlockSpec((B, DS), lambda i: (i, 0)),
        out_shape=jax.ShapeDtypeStruct((N, DS), jnp.float32),
    )(s, s_out, W_s, bs, g, b)


def kernel(s, v, edge_index, edge_attr, edge_vec_unit,
           W_e1, b_e1, W_e2, b_e2, W_s, b_s, W_v, b_v, ln_g, ln_b):
    row = edge_index[0]
    col = edge_index[1]
    W1a = W_e1[:, :DS]
    W1b = W_e1[:, DS:2 * DS]
    W1c = W_e1[:, 2 * DS:]
    WvT = W_v.T
    jidx = jnp.arange(96) // 3
    Wv_int = jnp.concatenate(
        [WvT[:, jidx], jnp.zeros((DS, DV), jnp.float32)], axis=1)
    bv_int = jnp.pad(b_v[jidx], (0, DV)).reshape(1, DS)
    kcol = jnp.arange(96) % 3
    P = jnp.pad((kcol[None, :] == jnp.arange(3)[:, None]).astype(jnp.float32),
                ((0, 0), (0, DV)))

    sa, sb = _node_proj(s, W1a, W1b)
    mlp_w = (W1c, b_e1.reshape(1, DS), W_e2, b_e2.reshape(1, DS),
             Wv_int, bv_int, P)
    g0 = _gather(sa, sb, row, col, 0, E0)
    g1 = _gather(sa, sb, row, col, E0, E1)
    m0, w0 = _edge_mlp(g0, edge_attr, edge_vec_unit, *mlp_w, 0, E0, E0)
    m1, w1 = _edge_mlp(g1, edge_attr, edge_vec_unit, *mlp_w,
                       E0 // 1024, E1, EPB)
    rowp = jnp.concatenate(
        [row[:E0], jnp.pad(row[E0:], (0, EPB - E1), mode="wrap")])
    v2p = jnp.pad(v.reshape(N, 3 * DV), ((0, NP - N), (0, 0)))
    s_out_p, v_new96p = _scatter(m0, w0, m1, w1, rowp, v2p)
    s_new = _node_out(s, s_out_p[:N], W_s, b_s.reshape(1, DS),
                      ln_g.reshape(1, DS), ln_b.reshape(1, DS))
    return (s_new, v_new96p[:N].reshape(N, DV, 3))
